# baseline (speedup 1.0000x reference)
import jax
import jax.numpy as jnp
from jax.experimental import pallas as pl
from jax.experimental.pallas import tpu as pltpu

_IN = 768
_E = 8
_D = 128
_B = 2048
_TB = 1024


def _moe_body(x_ref, wg_ref, w1_ref, b1_ref, w2_ref, b2_ref,
              out_ref, gw_ref, xb_ref, wrep_ref):
    e = pl.program_id(1)

    @pl.when(e == 0)
    def _prep():
        x = x_ref[...]
        logits = jax.lax.dot_general(
            x, wg_ref[...], (((1,), (1,)), ((), ())),
            preferred_element_type=jnp.float32)
        m1 = jnp.max(logits, axis=-1, keepdims=True)
        masked = jnp.where(logits == m1, -jnp.inf, logits)
        m2 = jnp.max(masked, axis=-1, keepdims=True)
        denom = 1.0 + jnp.exp(m2 - m1)
        gw = jnp.where(logits >= m2, jnp.exp(logits - m1), 0.0) / denom
        gw_ref[...] = gw
        xb_ref[...] = x.astype(jnp.bfloat16)
        sel = (jax.lax.broadcasted_iota(jnp.int32, (_E, _E * _D), 0)
               == jax.lax.broadcasted_iota(jnp.int32, (_E, _E * _D), 1) // _D
               ).astype(jnp.bfloat16)
        wrep_ref[...] = jnp.dot(gw.astype(jnp.bfloat16), sel,
                                preferred_element_type=jnp.float32)

    w1b = w1_ref[0].astype(jnp.bfloat16)
    h = jax.lax.dot_general(
        xb_ref[...], w1b, (((1,), (1,)), ((), ())),
        preferred_element_type=jnp.float32) + b1_ref[0]
    h = 0.5 * h * (1.0 + jax.lax.erf(h * 0.7071067811865476))
    o_e = jax.lax.dot_general(
        h.astype(jnp.bfloat16), w2_ref[0].astype(jnp.bfloat16),
        (((1,), (1,)), ((), ())),
        preferred_element_type=jnp.float32)
    ge = wrep_ref[:, pl.ds(e * _D, _D)]
    contrib = ge * (o_e + b2_ref[0])

    @pl.when(e == 0)
    def _init():
        out_ref[...] = contrib

    @pl.when(e != 0)
    def _acc():
        out_ref[...] = out_ref[...] + contrib


@jax.jit
def kernel(x, Wg, W1, b1, W2, b2):
    grid = (_B // _TB, _E)
    out, gw = pl.pallas_call(
        _moe_body,
        grid=grid,
        in_specs=[
            pl.BlockSpec((_TB, _IN), lambda i, e: (i, 0)),
            pl.BlockSpec((_E, _IN), lambda i, e: (0, 0)),
            pl.BlockSpec((1, _D, _IN), lambda i, e: (e, 0, 0)),
            pl.BlockSpec((1, 1, _D), lambda i, e: (e, 0, 0)),
            pl.BlockSpec((1, _D, _D), lambda i, e: (e, 0, 0)),
            pl.BlockSpec((1, 1, _D), lambda i, e: (e, 0, 0)),
        ],
        out_specs=[
            pl.BlockSpec((_TB, _D), lambda i, e: (i, 0)),
            pl.BlockSpec((_TB, _E), lambda i, e: (i, 0)),
        ],
        out_shape=[
            jax.ShapeDtypeStruct((_B, _D), jnp.float32),
            jax.ShapeDtypeStruct((_B, _E), jnp.float32),
        ],
        scratch_shapes=[
            pltpu.VMEM((_TB, _IN), jnp.bfloat16),
            pltpu.VMEM((_TB, _E * _D), jnp.float32),
        ],
    )(x, Wg, W1, b1.reshape(_E, 1, _D), W2, b2.reshape(_E, 1, _D))
    return out, gw


# speedup vs baseline: 1.8641x; 1.8641x over previous
import jax
import jax.numpy as jnp
from jax.experimental import pallas as pl
from jax.experimental.pallas import tpu as pltpu

_IN = 768
_E = 8
_D = 128
_B = 2048
_TB = 1024


def _moe_body(x_ref, wg_ref, w1_ref, b1_ref, w2_ref, b2_ref,
              out_ref, gw_ref, w1b_ref, w2b_ref, sel_ref):
    i = pl.program_id(0)

    @pl.when(i == 0)
    def _prep():
        w1b_ref[...] = w1_ref[...].astype(jnp.bfloat16)
        w2b_ref[...] = (jnp.transpose(w2_ref[...], (0, 2, 1))
                        .reshape(_E * _D, _D).astype(jnp.bfloat16))
        sel_ref[...] = (
            jax.lax.broadcasted_iota(jnp.int32, (_E, _E * _D), 0)
            == jax.lax.broadcasted_iota(jnp.int32, (_E, _E * _D), 1) // _D
        ).astype(jnp.bfloat16)

    x = x_ref[...]
    logits = jax.lax.dot_general(
        x, wg_ref[...], (((1,), (1,)), ((), ())),
        preferred_element_type=jnp.float32)
    m1 = jnp.max(logits, axis=-1, keepdims=True)
    masked = jnp.where(logits == m1, -jnp.inf, logits)
    m2 = jnp.max(masked, axis=-1, keepdims=True)
    denom = 1.0 + jnp.exp(m2 - m1)
    gw = jnp.where(logits >= m2, jnp.exp(logits - m1), 0.0) / denom
    gw_ref[...] = gw

    xb = x.astype(jnp.bfloat16)
    h = jax.lax.dot_general(
        xb, w1b_ref[...], (((1,), (1,)), ((), ())),
        preferred_element_type=jnp.float32) + b1_ref[...]
    h = 0.5 * h * (1.0 + jax.lax.erf(h * 0.7071067811865476))
    w_rep = jnp.dot(gw.astype(jnp.bfloat16), sel_ref[...],
                    preferred_element_type=jnp.float32)
    hw = (h * w_rep).astype(jnp.bfloat16)
    out = jnp.dot(hw, w2b_ref[...], preferred_element_type=jnp.float32)
    out_ref[...] = out + jnp.dot(gw, b2_ref[...],
                                 preferred_element_type=jnp.float32)


@jax.jit
def kernel(x, Wg, W1, b1, W2, b2):
    w1r = W1.reshape(_E * _D, _IN)
    b1f = b1.reshape(1, _E * _D)
    grid = (_B // _TB,)
    const2 = lambda i: (0, 0)
    const3 = lambda i: (0, 0, 0)
    out, gw = pl.pallas_call(
        _moe_body,
        grid=grid,
        in_specs=[
            pl.BlockSpec((_TB, _IN), lambda i: (i, 0)),
            pl.BlockSpec((_E, _IN), const2),
            pl.BlockSpec((_E * _D, _IN), const2),
            pl.BlockSpec((1, _E * _D), const2),
            pl.BlockSpec((_E, _D, _D), const3),
            pl.BlockSpec((_E, _D), const2),
        ],
        out_specs=[
            pl.BlockSpec((_TB, _D), lambda i: (i, 0)),
            pl.BlockSpec((_TB, _E), lambda i: (i, 0)),
        ],
        out_shape=[
            jax.ShapeDtypeStruct((_B, _D), jnp.float32),
            jax.ShapeDtypeStruct((_B, _E), jnp.float32),
        ],
        scratch_shapes=[
            pltpu.VMEM((_E * _D, _IN), jnp.bfloat16),
            pltpu.VMEM((_E * _D, _D), jnp.bfloat16),
            pltpu.VMEM((_E, _E * _D), jnp.bfloat16),
        ],
    )(x, Wg, w1r, b1f, W2, b2)
    return out, gw
